# baseline (speedup 1.0000x reference)
import functools

import jax
import jax.numpy as jnp
from jax import lax
from jax.experimental import pallas as pl
from jax.experimental.pallas import tpu as pltpu
from jax.experimental.pallas import tpu_sc as plsc

_NC = 2
_NS = 16
_LW = 128
_CW = 32
_NR = 8


def _segsum_body(m, n, npad, d, nhalf, xst, zs,
                 x_hbm, sd0_hbm, sd1_hbm, zrow_hbm,
                 part_hbm,
                 sd_t0, sd_t1, rows_a, rows_b,
                 sem_a, sem_b, sem_sa, sem_sb, sem_r0, sem_r1,
                 acc_sh, xsp_sh):
  c = lax.axis_index("c")
  s = lax.axis_index("s")
  wid = c * _NS + s

  zlast = npad - (_NS - 1) * zs
  last = nhalf - (_NS - 1) * xst

  @pl.when(s < _NS - 1)
  def _():
    pltpu.sync_copy(zrow_hbm.at[pl.ds(0, zs)],
                    acc_sh.at[pl.ds(s * zs, zs)])

  @pl.when(s == _NS - 1)
  def _():
    pltpu.sync_copy(zrow_hbm.at[pl.ds(0, zlast)],
                    acc_sh.at[pl.ds((_NS - 1) * zs, zlast)])

  def gath(sd, k, rows, sem):
    pltpu.async_copy(xsp_sh.at[sd.at[k, pl.ds(_CW, _CW)]], rows, sem)

  def wg(sd, k, rows, sem):
    pltpu.make_async_copy(xsp_sh.at[sd.at[k, pl.ds(_CW, _CW)]],
                          rows, sem).wait()

  def scat(sd, k, rows, sem):
    pltpu.async_copy(rows, acc_sh.at[sd.at[k, pl.ds(0, _CW)]], sem, add=True)

  def ws(sd, k, rows, sem):
    pltpu.make_async_copy(rows, acc_sh.at[sd.at[k, pl.ds(0, _CW)]],
                          sem).wait()

  def process(sd):
    def pair(t, carry):
      ka = 2 * t
      kb = 2 * t + 1
      wg(sd, ka, rows_a, sem_a)
      scat(sd, ka, rows_a, sem_sa)
      wg(sd, kb, rows_b, sem_b)
      scat(sd, kb, rows_b, sem_sb)
      ws(sd, ka, rows_a, sem_sa)

      @pl.when(ka + 2 < _NR)
      def _():
        gath(sd, ka + 2, rows_a, sem_a)

      ws(sd, kb, rows_b, sem_sb)

      @pl.when(kb + 2 < _NR)
      def _():
        gath(sd, kb + 2, rows_b, sem_b)

      return carry

    gath(sd, 0, rows_a, sem_a)
    gath(sd, 1, rows_b, sem_b)
    lax.fori_loop(0, _NR // 2, pair, 0)

  for h, sd_hbm in ((0, sd0_hbm), (1, sd1_hbm)):
    @pl.when(s < _NS - 1)
    def _():
      pltpu.sync_copy(x_hbm.at[pl.ds(h * nhalf + s * xst, xst)],
                      xsp_sh.at[pl.ds(s * xst, xst)])

    @pl.when(s == _NS - 1)
    def _():
      pltpu.sync_copy(x_hbm.at[pl.ds(h * nhalf + (_NS - 1) * xst, last)],
                      xsp_sh.at[pl.ds((_NS - 1) * xst, last)])

    plsc.subcore_barrier()

    nring = m // _NR

    def stage(sd, r, sem):
      pltpu.async_copy(sd_hbm.at[pl.ds(wid * m + r * _NR, _NR)], sd, sem)

    def wstage(sd, r, sem):
      pltpu.make_async_copy(sd_hbm.at[pl.ds(wid * m + r * _NR, _NR)],
                            sd, sem).wait()

    def ringpair(rr, carry):
      r0 = 2 * rr
      wstage(sd_t0, r0, sem_r0)
      stage(sd_t1, r0 + 1, sem_r1)
      process(sd_t0)
      wstage(sd_t1, r0 + 1, sem_r1)

      @pl.when(r0 + 2 < nring)
      def _():
        stage(sd_t0, r0 + 2, sem_r0)

      process(sd_t1)
      return carry

    stage(sd_t0, 0, sem_r0)
    lax.fori_loop(0, nring // 2, ringpair, 0)
    plsc.subcore_barrier()

  @pl.when(s < _NS - 1)
  def _():
    pltpu.sync_copy(acc_sh.at[pl.ds(s * zs, zs)],
                    part_hbm.at[c, pl.ds(s * zs, zs)])

  @pl.when(s == _NS - 1)
  def _():
    pltpu.sync_copy(acc_sh.at[pl.ds((_NS - 1) * zs, zlast)],
                    part_hbm.at[c, pl.ds((_NS - 1) * zs, zlast)])


@functools.partial(jax.jit, static_argnums=(3, 4))
def _sc_segsum(x, sd0, sd1, n, d):
  nt = _NC * _NS
  m = sd0.shape[0] // nt
  npad = n + 8
  nhalf = n // 2
  xst = (-(-nhalf // _NS) + 7) // 8 * 8
  zs = (-(-npad // _NS) + 7) // 8 * 8
  assert n % 16 == 0 and (_NS - 1) * xst < nhalf and (_NS - 1) * zs < npad
  assert (nhalf - (_NS - 1) * xst) % 8 == 0 and (npad - (_NS - 1) * zs) % 8 == 0

  zrow = jnp.zeros((zs, d), jnp.float32)

  mesh = plsc.VectorSubcoreMesh(core_axis_name="c", subcore_axis_name="s")
  fn = pl.kernel(
      functools.partial(_segsum_body, m, n, npad, d, nhalf, xst, zs),
      out_type=jax.ShapeDtypeStruct((_NC, npad, d), jnp.float32),
      mesh=mesh,
      scratch_types=[
          pltpu.VMEM((_NR, 2 * _CW), jnp.int32),
          pltpu.VMEM((_NR, 2 * _CW), jnp.int32),
          pltpu.VMEM((_CW, d), jnp.float32),
          pltpu.VMEM((_CW, d), jnp.float32),
          pltpu.SemaphoreType.DMA,
          pltpu.SemaphoreType.DMA,
          pltpu.SemaphoreType.DMA,
          pltpu.SemaphoreType.DMA,
          pltpu.SemaphoreType.DMA,
          pltpu.SemaphoreType.DMA,
          pltpu.VMEM_SHARED((npad, d), jnp.float32),
          pltpu.VMEM_SHARED((nhalf, d), jnp.float32),
      ],
  )
  return fn(x, sd0, sd1, zrow)


def _count_body(nch, npad, d,
                dstp_hbm, zcnt_hbm, ones_hbm,
                cntp_hbm,
                dst_t, ones_v, cnt_sh):
  c = lax.axis_index("c")
  s = lax.axis_index("s")
  wid = c * _NS + s
  zrows = npad // _NS

  pltpu.sync_copy(dstp_hbm.at[pl.ds(wid * nch, nch)], dst_t)
  pltpu.sync_copy(ones_hbm, ones_v)
  pltpu.sync_copy(zcnt_hbm, cnt_sh.at[pl.ds(s * zrows, zrows)])
  plsc.subcore_barrier()

  def chunk(j, carry):
    pltpu.sync_copy(ones_v, cnt_sh.at[dst_t.at[j]], add=True)
    return carry

  lax.fori_loop(0, nch, chunk, 0)
  plsc.subcore_barrier()
  pltpu.sync_copy(cnt_sh.at[pl.ds(s * zrows, zrows)],
                  cntp_hbm.at[c, pl.ds(s * zrows, zrows)])


@functools.partial(jax.jit, static_argnums=(1, 2))
def _sc_counts(dstp, n, d):
  nt = _NC * _NS
  nch = dstp.shape[0] // nt
  npad = (n + 128) // 128 * 128
  zcnt = jnp.zeros((npad // _NS, d), jnp.float32)
  ones = jnp.ones((_LW, d), jnp.float32)
  mesh = plsc.VectorSubcoreMesh(core_axis_name="c", subcore_axis_name="s")
  fn = pl.kernel(
      functools.partial(_count_body, nch, npad, d),
      out_type=jax.ShapeDtypeStruct((_NC, npad, d), jnp.float32),
      mesh=mesh,
      scratch_types=[
          pltpu.VMEM((nch, _LW), jnp.int32),
          pltpu.VMEM((_LW, d), jnp.float32),
          pltpu.VMEM_SHARED((npad, d), jnp.float32),
      ],
  )
  return fn(dstp, zcnt, ones)


def _phase_body(final, p_ref, cnt_ref, x_ref, wl_ref, wr_ref, b_ref, o_ref):
  s = p_ref[0] + p_ref[1]
  cnt = cnt_ref[0][:, :1] + cnt_ref[1][:, :1]
  mean = s / jnp.maximum(cnt, 1.0)
  h = (jnp.dot(mean, wl_ref[...], preferred_element_type=jnp.float32)
       + jnp.dot(x_ref[...], wr_ref[...], preferred_element_type=jnp.float32)
       + b_ref[...])
  if final:
    nrm = jnp.sqrt(jnp.sum(h * h, axis=1, keepdims=True))
    o_ref[...] = h / jnp.maximum(nrm, 1e-12)
  else:
    o_ref[...] = jnp.where(h >= 0, h, 0.01 * h)


@functools.partial(jax.jit, static_argnums=(6,))
def _tc_phase(part, cntp, x, wlt, wrt, b, final):
  n, d = x.shape
  r = 1000
  grid = (n // r,)
  return pl.pallas_call(
      functools.partial(_phase_body, final),
      grid=grid,
      in_specs=[
          pl.BlockSpec((_NC, r, d), lambda i: (0, i, 0)),
          pl.BlockSpec((_NC, r, d), lambda i: (0, i, 0)),
          pl.BlockSpec((r, d), lambda i: (i, 0)),
          pl.BlockSpec((d, d), lambda i: (0, 0)),
          pl.BlockSpec((d, d), lambda i: (0, 0)),
          pl.BlockSpec((1, d), lambda i: (0, 0)),
      ],
      out_specs=pl.BlockSpec((r, d), lambda i: (i, 0)),
      out_shape=jax.ShapeDtypeStruct((n, d), jnp.float32),
  )(part, cntp, x, wlt, wrt, b)


def kernel(x, edge_indices, Wl1, Wr1, b1, Wl2, Wr2, b2):
  n, d = x.shape
  e = edge_indices.shape[1]
  nt = _NC * _NS
  nhalf = n // 2

  nch = -(-e // (_LW * nt))
  nch = (nch + 7) // 8 * 8
  epad = nch * _LW * nt
  dst128 = jnp.concatenate(
      [edge_indices[1], jnp.full((epad - e,), n, jnp.int32)]).reshape(-1, _LW)

  m = -(-e // (_CW * nt))
  m = (m + _NR - 1) // _NR * _NR
  ep2 = m * _CW * nt
  srcf = jnp.concatenate([edge_indices[0], jnp.full((ep2 - e,), n, jnp.int32)])
  dstf = jnp.concatenate([edge_indices[1], jnp.zeros((ep2 - e,), jnp.int32)])
  sd = []
  for h in (0, 1):
    lo = h * nhalf
    inh = (srcf >= lo) & (srcf < lo + nhalf)
    sp = jnp.where(inh, srcf - lo, 0).reshape(-1, 1, _CW)
    dp = jnp.where(inh, dstf, n).reshape(-1, 1, _CW)
    sd.append(jnp.concatenate([dp, sp], axis=1).reshape(-1, 2 * _CW))

  cp = _sc_counts(dst128, n, d)
  p1 = _sc_segsum(x, sd[0], sd[1], n, d)
  h = _tc_phase(p1, cp, x, Wl1.T, Wr1.T, b1.reshape(1, d), False)
  p2 = _sc_segsum(h, sd[0], sd[1], n, d)
  return _tc_phase(p2, cp, h, Wl2.T, Wr2.T, b2.reshape(1, d), True)


# speedup vs baseline: 1.1352x; 1.1352x over previous
import functools

import jax
import jax.numpy as jnp
from jax import lax
from jax.experimental import pallas as pl
from jax.experimental.pallas import tpu as pltpu
from jax.experimental.pallas import tpu_sc as plsc

_NC = 2
_NS = 16
_LW = 128
_CW = 32
_NR = 8


def _segsum_body(m, n, npad, d, nhalf, xst, zs,
                 x_hbm, sd0_hbm, sd1_hbm, zrow_hbm,
                 part_hbm,
                 sd_t0, sd_t1, rows_a, rows_b,
                 sem_a, sem_b, sem_sa, sem_sb, sem_r0, sem_r1,
                 acc_sh, xsp_sh):
  c = lax.axis_index("c")
  s = lax.axis_index("s")
  wid = c * _NS + s

  zlast = npad - (_NS - 1) * zs
  last = nhalf - (_NS - 1) * xst

  @pl.when(s < _NS - 1)
  def _():
    pltpu.sync_copy(zrow_hbm.at[pl.ds(0, zs)],
                    acc_sh.at[pl.ds(s * zs, zs)])

  @pl.when(s == _NS - 1)
  def _():
    pltpu.sync_copy(zrow_hbm.at[pl.ds(0, zlast)],
                    acc_sh.at[pl.ds((_NS - 1) * zs, zlast)])

  def gath(sd, k, rows, sem):
    pltpu.async_copy(xsp_sh.at[sd.at[k, pl.ds(_CW, _CW)]], rows, sem)

  def wg(sd, k, rows, sem):
    pltpu.make_async_copy(xsp_sh.at[sd.at[k, pl.ds(_CW, _CW)]],
                          rows, sem).wait()

  def scat(sd, k, rows, sem):
    pltpu.async_copy(rows, acc_sh.at[sd.at[k, pl.ds(0, _CW)]], sem, add=True)

  def ws(sd, k, rows, sem):
    pltpu.make_async_copy(rows, acc_sh.at[sd.at[k, pl.ds(0, _CW)]],
                          sem).wait()

  def process(sd):
    def pair(t, carry):
      ka = 2 * t
      kb = 2 * t + 1
      wg(sd, ka, rows_a, sem_a)
      pltpu.sync_copy(rows_a, acc_sh.at[sd.at[ka, pl.ds(0, _CW)]], add=True)

      @pl.when(ka + 2 < _NR)
      def _():
        gath(sd, ka + 2, rows_a, sem_a)

      wg(sd, kb, rows_b, sem_b)
      pltpu.sync_copy(rows_b, acc_sh.at[sd.at[kb, pl.ds(0, _CW)]], add=True)

      @pl.when(kb + 2 < _NR)
      def _():
        gath(sd, kb + 2, rows_b, sem_b)

      return carry

    gath(sd, 0, rows_a, sem_a)
    gath(sd, 1, rows_b, sem_b)
    lax.fori_loop(0, _NR // 2, pair, 0)

  for h, sd_hbm in ((0, sd0_hbm), (1, sd1_hbm)):
    @pl.when(s < _NS - 1)
    def _():
      pltpu.sync_copy(x_hbm.at[pl.ds(h * nhalf + s * xst, xst)],
                      xsp_sh.at[pl.ds(s * xst, xst)])

    @pl.when(s == _NS - 1)
    def _():
      pltpu.sync_copy(x_hbm.at[pl.ds(h * nhalf + (_NS - 1) * xst, last)],
                      xsp_sh.at[pl.ds((_NS - 1) * xst, last)])

    plsc.subcore_barrier()

    nring = m // _NR

    def stage(sd, r, sem):
      pltpu.async_copy(sd_hbm.at[pl.ds(wid * m + r * _NR, _NR)], sd, sem)

    def wstage(sd, r, sem):
      pltpu.make_async_copy(sd_hbm.at[pl.ds(wid * m + r * _NR, _NR)],
                            sd, sem).wait()

    def ringpair(rr, carry):
      r0 = 2 * rr
      wstage(sd_t0, r0, sem_r0)
      stage(sd_t1, r0 + 1, sem_r1)
      process(sd_t0)
      wstage(sd_t1, r0 + 1, sem_r1)

      @pl.when(r0 + 2 < nring)
      def _():
        stage(sd_t0, r0 + 2, sem_r0)

      process(sd_t1)
      return carry

    stage(sd_t0, 0, sem_r0)
    lax.fori_loop(0, nring // 2, ringpair, 0)
    plsc.subcore_barrier()

  @pl.when(s < _NS - 1)
  def _():
    pltpu.sync_copy(acc_sh.at[pl.ds(s * zs, zs)],
                    part_hbm.at[c, pl.ds(s * zs, zs)])

  @pl.when(s == _NS - 1)
  def _():
    pltpu.sync_copy(acc_sh.at[pl.ds((_NS - 1) * zs, zlast)],
                    part_hbm.at[c, pl.ds((_NS - 1) * zs, zlast)])


@functools.partial(jax.jit, static_argnums=(3, 4))
def _sc_segsum(x, sd0, sd1, n, d):
  nt = _NC * _NS
  m = sd0.shape[0] // nt
  npad = n + 8
  nhalf = n // 2
  xst = (-(-nhalf // _NS) + 7) // 8 * 8
  zs = (-(-npad // _NS) + 7) // 8 * 8
  assert n % 16 == 0 and (_NS - 1) * xst < nhalf and (_NS - 1) * zs < npad
  assert (nhalf - (_NS - 1) * xst) % 8 == 0 and (npad - (_NS - 1) * zs) % 8 == 0

  zrow = jnp.zeros((zs, d), jnp.float32)

  mesh = plsc.VectorSubcoreMesh(core_axis_name="c", subcore_axis_name="s")
  fn = pl.kernel(
      functools.partial(_segsum_body, m, n, npad, d, nhalf, xst, zs),
      out_type=jax.ShapeDtypeStruct((_NC, npad, d), jnp.float32),
      mesh=mesh,
      scratch_types=[
          pltpu.VMEM((_NR, 2 * _CW), jnp.int32),
          pltpu.VMEM((_NR, 2 * _CW), jnp.int32),
          pltpu.VMEM((_CW, d), jnp.float32),
          pltpu.VMEM((_CW, d), jnp.float32),
          pltpu.SemaphoreType.DMA,
          pltpu.SemaphoreType.DMA,
          pltpu.SemaphoreType.DMA,
          pltpu.SemaphoreType.DMA,
          pltpu.SemaphoreType.DMA,
          pltpu.SemaphoreType.DMA,
          pltpu.VMEM_SHARED((npad, d), jnp.float32),
          pltpu.VMEM_SHARED((nhalf, d), jnp.float32),
      ],
  )
  return fn(x, sd0, sd1, zrow)


def _count_body(nch, npad, d,
                dstp_hbm, zcnt_hbm, ones_hbm,
                cntp_hbm,
                dst_t, ones_v, cnt_sh):
  c = lax.axis_index("c")
  s = lax.axis_index("s")
  wid = c * _NS + s
  zrows = npad // _NS

  pltpu.sync_copy(dstp_hbm.at[pl.ds(wid * nch, nch)], dst_t)
  pltpu.sync_copy(ones_hbm, ones_v)
  pltpu.sync_copy(zcnt_hbm, cnt_sh.at[pl.ds(s * zrows, zrows)])
  plsc.subcore_barrier()

  def chunk(j, carry):
    pltpu.sync_copy(ones_v, cnt_sh.at[dst_t.at[j]], add=True)
    return carry

  lax.fori_loop(0, nch, chunk, 0)
  plsc.subcore_barrier()
  pltpu.sync_copy(cnt_sh.at[pl.ds(s * zrows, zrows)],
                  cntp_hbm.at[c, pl.ds(s * zrows, zrows)])


@functools.partial(jax.jit, static_argnums=(1, 2))
def _sc_counts(dstp, n, d):
  nt = _NC * _NS
  nch = dstp.shape[0] // nt
  npad = (n + 128) // 128 * 128
  zcnt = jnp.zeros((npad // _NS, d), jnp.float32)
  ones = jnp.ones((_LW, d), jnp.float32)
  mesh = plsc.VectorSubcoreMesh(core_axis_name="c", subcore_axis_name="s")
  fn = pl.kernel(
      functools.partial(_count_body, nch, npad, d),
      out_type=jax.ShapeDtypeStruct((_NC, npad, d), jnp.float32),
      mesh=mesh,
      scratch_types=[
          pltpu.VMEM((nch, _LW), jnp.int32),
          pltpu.VMEM((_LW, d), jnp.float32),
          pltpu.VMEM_SHARED((npad, d), jnp.float32),
      ],
  )
  return fn(dstp, zcnt, ones)


def _phase_body(final, p_ref, cnt_ref, x_ref, wl_ref, wr_ref, b_ref, o_ref):
  s = p_ref[0] + p_ref[1]
  cnt = cnt_ref[0][:, :1] + cnt_ref[1][:, :1]
  mean = s / jnp.maximum(cnt, 1.0)
  h = (jnp.dot(mean, wl_ref[...], preferred_element_type=jnp.float32)
       + jnp.dot(x_ref[...], wr_ref[...], preferred_element_type=jnp.float32)
       + b_ref[...])
  if final:
    nrm = jnp.sqrt(jnp.sum(h * h, axis=1, keepdims=True))
    o_ref[...] = h / jnp.maximum(nrm, 1e-12)
  else:
    o_ref[...] = jnp.where(h >= 0, h, 0.01 * h)


@functools.partial(jax.jit, static_argnums=(6,))
def _tc_phase(part, cntp, x, wlt, wrt, b, final):
  n, d = x.shape
  r = 1000
  grid = (n // r,)
  return pl.pallas_call(
      functools.partial(_phase_body, final),
      grid=grid,
      in_specs=[
          pl.BlockSpec((_NC, r, d), lambda i: (0, i, 0)),
          pl.BlockSpec((_NC, r, d), lambda i: (0, i, 0)),
          pl.BlockSpec((r, d), lambda i: (i, 0)),
          pl.BlockSpec((d, d), lambda i: (0, 0)),
          pl.BlockSpec((d, d), lambda i: (0, 0)),
          pl.BlockSpec((1, d), lambda i: (0, 0)),
      ],
      out_specs=pl.BlockSpec((r, d), lambda i: (i, 0)),
      out_shape=jax.ShapeDtypeStruct((n, d), jnp.float32),
  )(part, cntp, x, wlt, wrt, b)


def kernel(x, edge_indices, Wl1, Wr1, b1, Wl2, Wr2, b2):
  n, d = x.shape
  e = edge_indices.shape[1]
  nt = _NC * _NS
  nhalf = n // 2

  nch = -(-e // (_LW * nt))
  nch = (nch + 7) // 8 * 8
  epad = nch * _LW * nt
  dst128 = jnp.concatenate(
      [edge_indices[1], jnp.full((epad - e,), n, jnp.int32)]).reshape(-1, _LW)

  m = -(-e // (_CW * nt))
  m = (m + _NR - 1) // _NR * _NR
  ep2 = m * _CW * nt
  srcf = jnp.concatenate([edge_indices[0], jnp.full((ep2 - e,), n, jnp.int32)])
  dstf = jnp.concatenate([edge_indices[1], jnp.zeros((ep2 - e,), jnp.int32)])
  sd = []
  for h in (0, 1):
    lo = h * nhalf
    inh = (srcf >= lo) & (srcf < lo + nhalf)
    sp = jnp.where(inh, srcf - lo, 0).reshape(-1, 1, _CW)
    dp = jnp.where(inh, dstf, n).reshape(-1, 1, _CW)
    sd.append(jnp.concatenate([dp, sp], axis=1).reshape(-1, 2 * _CW))

  cp = _sc_counts(dst128, n, d)
  p1 = _sc_segsum(x, sd[0], sd[1], n, d)
  h = _tc_phase(p1, cp, x, Wl1.T, Wr1.T, b1.reshape(1, d), False)
  p2 = _sc_segsum(h, sd[0], sd[1], n, d)
  return _tc_phase(p2, cp, h, Wl2.T, Wr2.T, b2.reshape(1, d), True)


# speedup vs baseline: 1.1354x; 1.0001x over previous
import functools

import jax
import jax.numpy as jnp
from jax import lax
from jax.experimental import pallas as pl
from jax.experimental.pallas import tpu as pltpu
from jax.experimental.pallas import tpu_sc as plsc

_NC = 2
_NS = 16
_LW = 128
_CW = 32
_NR = 8


def _segsum_body(m, n, npad, d, nhalf, xst, zs,
                 x_hbm, sd0_hbm, sd1_hbm, zrow_hbm,
                 part_hbm,
                 sd_t0, sd_t1, rows_a, rows_b,
                 sem_a, sem_b, sem_sa, sem_sb, sem_r0, sem_r1,
                 acc_sh, xsp_sh):
  c = lax.axis_index("c")
  s = lax.axis_index("s")
  wid = c * _NS + s

  zlast = npad - (_NS - 1) * zs
  last = nhalf - (_NS - 1) * xst

  @pl.when(s < _NS - 1)
  def _():
    pltpu.sync_copy(zrow_hbm.at[pl.ds(0, zs)],
                    acc_sh.at[pl.ds(s * zs, zs)])

  @pl.when(s == _NS - 1)
  def _():
    pltpu.sync_copy(zrow_hbm.at[pl.ds(0, zlast)],
                    acc_sh.at[pl.ds((_NS - 1) * zs, zlast)])

  def gath(sd, k, rows, sem):
    pltpu.async_copy(xsp_sh.at[sd.at[k, pl.ds(_CW, _CW)]], rows, sem)

  def wg(sd, k, rows, sem):
    pltpu.make_async_copy(zrow_hbm.at[pl.ds(0, _CW)], rows, sem).wait()

  def scat(sd, k, rows, sem):
    pltpu.async_copy(rows, acc_sh.at[sd.at[k, pl.ds(0, _CW)]], sem, add=True)

  def ws(sd, k, rows, sem):
    pltpu.make_async_copy(rows, acc_sh.at[sd.at[k, pl.ds(0, _CW)]],
                          sem).wait()

  def process(sd):
    def pair(t, carry):
      ka = 2 * t
      kb = 2 * t + 1
      wg(sd, ka, rows_a, sem_a)
      pltpu.sync_copy(rows_a, acc_sh.at[sd.at[ka, pl.ds(0, _CW)]], add=True)

      @pl.when(ka + 2 < _NR)
      def _():
        gath(sd, ka + 2, rows_a, sem_a)

      wg(sd, kb, rows_b, sem_b)
      pltpu.sync_copy(rows_b, acc_sh.at[sd.at[kb, pl.ds(0, _CW)]], add=True)

      @pl.when(kb + 2 < _NR)
      def _():
        gath(sd, kb + 2, rows_b, sem_b)

      return carry

    gath(sd, 0, rows_a, sem_a)
    gath(sd, 1, rows_b, sem_b)
    lax.fori_loop(0, _NR // 2, pair, 0)

  for h, sd_hbm in ((0, sd0_hbm), (1, sd1_hbm)):
    @pl.when(s < _NS - 1)
    def _():
      pltpu.sync_copy(x_hbm.at[pl.ds(h * nhalf + s * xst, xst)],
                      xsp_sh.at[pl.ds(s * xst, xst)])

    @pl.when(s == _NS - 1)
    def _():
      pltpu.sync_copy(x_hbm.at[pl.ds(h * nhalf + (_NS - 1) * xst, last)],
                      xsp_sh.at[pl.ds((_NS - 1) * xst, last)])

    plsc.subcore_barrier()

    nring = m // _NR

    def stage(sd, r, sem):
      pltpu.async_copy(sd_hbm.at[pl.ds(wid * m + r * _NR, _NR)], sd, sem)

    def wstage(sd, r, sem):
      pltpu.make_async_copy(sd_hbm.at[pl.ds(0, _NR)], sd, sem).wait()

    def ringpair(rr, carry):
      r0 = 2 * rr
      wstage(sd_t0, r0, sem_r0)
      stage(sd_t1, r0 + 1, sem_r1)
      process(sd_t0)
      wstage(sd_t1, r0 + 1, sem_r1)

      @pl.when(r0 + 2 < nring)
      def _():
        stage(sd_t0, r0 + 2, sem_r0)

      process(sd_t1)
      return carry

    stage(sd_t0, 0, sem_r0)
    lax.fori_loop(0, nring // 2, ringpair, 0)
    plsc.subcore_barrier()

  @pl.when(s < _NS - 1)
  def _():
    pltpu.sync_copy(acc_sh.at[pl.ds(s * zs, zs)],
                    part_hbm.at[c, pl.ds(s * zs, zs)])

  @pl.when(s == _NS - 1)
  def _():
    pltpu.sync_copy(acc_sh.at[pl.ds((_NS - 1) * zs, zlast)],
                    part_hbm.at[c, pl.ds((_NS - 1) * zs, zlast)])


@functools.partial(jax.jit, static_argnums=(3, 4))
def _sc_segsum(x, sd0, sd1, n, d):
  nt = _NC * _NS
  m = sd0.shape[0] // nt
  npad = n + 8
  nhalf = n // 2
  xst = (-(-nhalf // _NS) + 7) // 8 * 8
  zs = (-(-npad // _NS) + 7) // 8 * 8
  assert n % 16 == 0 and (_NS - 1) * xst < nhalf and (_NS - 1) * zs < npad
  assert (nhalf - (_NS - 1) * xst) % 8 == 0 and (npad - (_NS - 1) * zs) % 8 == 0

  zrow = jnp.zeros((zs, d), jnp.float32)

  mesh = plsc.VectorSubcoreMesh(core_axis_name="c", subcore_axis_name="s")
  fn = pl.kernel(
      functools.partial(_segsum_body, m, n, npad, d, nhalf, xst, zs),
      out_type=jax.ShapeDtypeStruct((_NC, npad, d), jnp.float32),
      mesh=mesh,
      scratch_types=[
          pltpu.VMEM((_NR, 2 * _CW), jnp.int32),
          pltpu.VMEM((_NR, 2 * _CW), jnp.int32),
          pltpu.VMEM((_CW, d), jnp.float32),
          pltpu.VMEM((_CW, d), jnp.float32),
          pltpu.SemaphoreType.DMA,
          pltpu.SemaphoreType.DMA,
          pltpu.SemaphoreType.DMA,
          pltpu.SemaphoreType.DMA,
          pltpu.SemaphoreType.DMA,
          pltpu.SemaphoreType.DMA,
          pltpu.VMEM_SHARED((npad, d), jnp.float32),
          pltpu.VMEM_SHARED((nhalf, d), jnp.float32),
      ],
  )
  return fn(x, sd0, sd1, zrow)


def _count_body(nch, npad, d,
                dstp_hbm, zcnt_hbm, ones_hbm,
                cntp_hbm,
                dst_t, ones_v, cnt_sh):
  c = lax.axis_index("c")
  s = lax.axis_index("s")
  wid = c * _NS + s
  zrows = npad // _NS

  pltpu.sync_copy(dstp_hbm.at[pl.ds(wid * nch, nch)], dst_t)
  pltpu.sync_copy(ones_hbm, ones_v)
  pltpu.sync_copy(zcnt_hbm, cnt_sh.at[pl.ds(s * zrows, zrows)])
  plsc.subcore_barrier()

  def chunk(j, carry):
    pltpu.sync_copy(ones_v, cnt_sh.at[dst_t.at[j]], add=True)
    return carry

  lax.fori_loop(0, nch, chunk, 0)
  plsc.subcore_barrier()
  pltpu.sync_copy(cnt_sh.at[pl.ds(s * zrows, zrows)],
                  cntp_hbm.at[c, pl.ds(s * zrows, zrows)])


@functools.partial(jax.jit, static_argnums=(1, 2))
def _sc_counts(dstp, n, d):
  nt = _NC * _NS
  nch = dstp.shape[0] // nt
  npad = (n + 128) // 128 * 128
  zcnt = jnp.zeros((npad // _NS, d), jnp.float32)
  ones = jnp.ones((_LW, d), jnp.float32)
  mesh = plsc.VectorSubcoreMesh(core_axis_name="c", subcore_axis_name="s")
  fn = pl.kernel(
      functools.partial(_count_body, nch, npad, d),
      out_type=jax.ShapeDtypeStruct((_NC, npad, d), jnp.float32),
      mesh=mesh,
      scratch_types=[
          pltpu.VMEM((nch, _LW), jnp.int32),
          pltpu.VMEM((_LW, d), jnp.float32),
          pltpu.VMEM_SHARED((npad, d), jnp.float32),
      ],
  )
  return fn(dstp, zcnt, ones)


def _phase_body(final, p_ref, cnt_ref, x_ref, wl_ref, wr_ref, b_ref, o_ref):
  s = p_ref[0] + p_ref[1]
  cnt = cnt_ref[0][:, :1] + cnt_ref[1][:, :1]
  mean = s / jnp.maximum(cnt, 1.0)
  h = (jnp.dot(mean, wl_ref[...], preferred_element_type=jnp.float32)
       + jnp.dot(x_ref[...], wr_ref[...], preferred_element_type=jnp.float32)
       + b_ref[...])
  if final:
    nrm = jnp.sqrt(jnp.sum(h * h, axis=1, keepdims=True))
    o_ref[...] = h / jnp.maximum(nrm, 1e-12)
  else:
    o_ref[...] = jnp.where(h >= 0, h, 0.01 * h)


@functools.partial(jax.jit, static_argnums=(6,))
def _tc_phase(part, cntp, x, wlt, wrt, b, final):
  n, d = x.shape
  r = 1000
  grid = (n // r,)
  return pl.pallas_call(
      functools.partial(_phase_body, final),
      grid=grid,
      in_specs=[
          pl.BlockSpec((_NC, r, d), lambda i: (0, i, 0)),
          pl.BlockSpec((_NC, r, d), lambda i: (0, i, 0)),
          pl.BlockSpec((r, d), lambda i: (i, 0)),
          pl.BlockSpec((d, d), lambda i: (0, 0)),
          pl.BlockSpec((d, d), lambda i: (0, 0)),
          pl.BlockSpec((1, d), lambda i: (0, 0)),
      ],
      out_specs=pl.BlockSpec((r, d), lambda i: (i, 0)),
      out_shape=jax.ShapeDtypeStruct((n, d), jnp.float32),
  )(part, cntp, x, wlt, wrt, b)


def kernel(x, edge_indices, Wl1, Wr1, b1, Wl2, Wr2, b2):
  n, d = x.shape
  e = edge_indices.shape[1]
  nt = _NC * _NS
  nhalf = n // 2

  nch = -(-e // (_LW * nt))
  nch = (nch + 7) // 8 * 8
  epad = nch * _LW * nt
  dst128 = jnp.concatenate(
      [edge_indices[1], jnp.full((epad - e,), n, jnp.int32)]).reshape(-1, _LW)

  m = -(-e // (_CW * nt))
  m = (m + _NR - 1) // _NR * _NR
  ep2 = m * _CW * nt
  srcf = jnp.concatenate([edge_indices[0], jnp.full((ep2 - e,), n, jnp.int32)])
  dstf = jnp.concatenate([edge_indices[1], jnp.zeros((ep2 - e,), jnp.int32)])
  sd = []
  for h in (0, 1):
    lo = h * nhalf
    inh = (srcf >= lo) & (srcf < lo + nhalf)
    sp = jnp.where(inh, srcf - lo, 0).reshape(-1, 1, _CW)
    dp = jnp.where(inh, dstf, n).reshape(-1, 1, _CW)
    sd.append(jnp.concatenate([dp, sp], axis=1).reshape(-1, 2 * _CW))

  cp = _sc_counts(dst128, n, d)
  p1 = _sc_segsum(x, sd[0], sd[1], n, d)
  h = _tc_phase(p1, cp, x, Wl1.T, Wr1.T, b1.reshape(1, d), False)
  p2 = _sc_segsum(h, sd[0], sd[1], n, d)
  return _tc_phase(p2, cp, h, Wl2.T, Wr2.T, b2.reshape(1, d), True)


# speedup vs baseline: 1.2662x; 1.1152x over previous
import functools

import jax
import jax.numpy as jnp
from jax import lax
from jax.experimental import pallas as pl
from jax.experimental.pallas import tpu as pltpu
from jax.experimental.pallas import tpu_sc as plsc

_NC = 2
_NS = 16
_LW = 128
_CW = 32
_NR = 8


def _segsum_body(m, n, npad, d, nhalf, xst, zs,
                 x_hbm, sd0_hbm, sd1_hbm, zrow_hbm,
                 part_hbm,
                 sd_t0, sd_t1, rows_a, rows_b,
                 sem_a, sem_b, sem_sa, sem_sb, sem_r0, sem_r1,
                 acc_sh, xsp_sh):
  c = lax.axis_index("c")
  s = lax.axis_index("s")
  wid = c * _NS + s

  zlast = npad - (_NS - 1) * zs
  last = nhalf - (_NS - 1) * xst

  @pl.when(s < _NS - 1)
  def _():
    pltpu.sync_copy(zrow_hbm.at[pl.ds(0, zs)],
                    acc_sh.at[pl.ds(s * zs, zs)])

  @pl.when(s == _NS - 1)
  def _():
    pltpu.sync_copy(zrow_hbm.at[pl.ds(0, zlast)],
                    acc_sh.at[pl.ds((_NS - 1) * zs, zlast)])

  def gath(sd, k, rows, sem):
    pltpu.async_copy(xsp_sh.at[sd.at[k, pl.ds(_CW, _CW)]], rows, sem)

  def wg(sd, k, rows, sem):
    pltpu.make_async_copy(zrow_hbm.at[pl.ds(0, _CW)], rows, sem).wait()

  def scat(sd, k, rows, sem):
    pltpu.async_copy(rows, acc_sh.at[sd.at[k, pl.ds(0, _CW)]], sem, add=True)

  def ws(sd, k, rows, sem):
    pltpu.make_async_copy(rows, acc_sh.at[sd.at[k, pl.ds(0, _CW)]],
                          sem).wait()

  def process(sd, sd_nxt, wait_nxt, nxt_pred):
    def pair(t, carry):
      ka = 2 * t
      kb = 2 * t + 1
      tail = ka + 2 >= _NR
      wg(sd, ka, rows_a, sem_a)
      pltpu.sync_copy(rows_a, acc_sh.at[sd.at[ka, pl.ds(0, _CW)]], add=True)

      @pl.when(jnp.logical_not(tail))
      def _():
        gath(sd, ka + 2, rows_a, sem_a)

      @pl.when(tail & nxt_pred)
      def _():
        wait_nxt()
        gath(sd_nxt, 0, rows_a, sem_a)

      wg(sd, kb, rows_b, sem_b)
      pltpu.sync_copy(rows_b, acc_sh.at[sd.at[kb, pl.ds(0, _CW)]], add=True)

      @pl.when(jnp.logical_not(tail))
      def _():
        gath(sd, kb + 2, rows_b, sem_b)

      @pl.when(tail & nxt_pred)
      def _():
        gath(sd_nxt, 1, rows_b, sem_b)

      return carry

    lax.fori_loop(0, _NR // 2, pair, 0)

  for h, sd_hbm in ((0, sd0_hbm), (1, sd1_hbm)):
    @pl.when(s < _NS - 1)
    def _():
      pltpu.sync_copy(x_hbm.at[pl.ds(h * nhalf + s * xst, xst)],
                      xsp_sh.at[pl.ds(s * xst, xst)])

    @pl.when(s == _NS - 1)
    def _():
      pltpu.sync_copy(x_hbm.at[pl.ds(h * nhalf + (_NS - 1) * xst, last)],
                      xsp_sh.at[pl.ds((_NS - 1) * xst, last)])

    plsc.subcore_barrier()

    nring = m // _NR

    def stage(sd, r, sem):
      pltpu.async_copy(sd_hbm.at[pl.ds(wid * m + r * _NR, _NR)], sd, sem)

    def wstage(sd, sem):
      pltpu.make_async_copy(sd_hbm.at[pl.ds(0, _NR)], sd, sem).wait()

    def ringpair(rr, carry):
      r0 = 2 * rr
      process(sd_t0, sd_t1, lambda: wstage(sd_t1, sem_r1), rr >= 0)

      @pl.when(r0 + 2 < nring)
      def _():
        stage(sd_t0, r0 + 2, sem_r0)

      process(sd_t1, sd_t0, lambda: wstage(sd_t0, sem_r0), r0 + 2 < nring)

      @pl.when(r0 + 3 < nring)
      def _():
        stage(sd_t1, r0 + 3, sem_r1)

      return carry

    stage(sd_t0, 0, sem_r0)
    wstage(sd_t0, sem_r0)
    stage(sd_t1, 1, sem_r1)
    gath(sd_t0, 0, rows_a, sem_a)
    gath(sd_t0, 1, rows_b, sem_b)
    lax.fori_loop(0, nring // 2, ringpair, 0)
    plsc.subcore_barrier()

  @pl.when(s < _NS - 1)
  def _():
    pltpu.sync_copy(acc_sh.at[pl.ds(s * zs, zs)],
                    part_hbm.at[c, pl.ds(s * zs, zs)])

  @pl.when(s == _NS - 1)
  def _():
    pltpu.sync_copy(acc_sh.at[pl.ds((_NS - 1) * zs, zlast)],
                    part_hbm.at[c, pl.ds((_NS - 1) * zs, zlast)])


@functools.partial(jax.jit, static_argnums=(3, 4))
def _sc_segsum(x, sd0, sd1, n, d):
  nt = _NC * _NS
  m = sd0.shape[0] // nt
  npad = n + 8
  nhalf = n // 2
  xst = (-(-nhalf // _NS) + 7) // 8 * 8
  zs = (-(-npad // _NS) + 7) // 8 * 8
  assert n % 16 == 0 and (_NS - 1) * xst < nhalf and (_NS - 1) * zs < npad
  assert (nhalf - (_NS - 1) * xst) % 8 == 0 and (npad - (_NS - 1) * zs) % 8 == 0

  zrow = jnp.zeros((zs, d), jnp.float32)

  mesh = plsc.VectorSubcoreMesh(core_axis_name="c", subcore_axis_name="s")
  fn = pl.kernel(
      functools.partial(_segsum_body, m, n, npad, d, nhalf, xst, zs),
      out_type=jax.ShapeDtypeStruct((_NC, npad, d), jnp.float32),
      mesh=mesh,
      scratch_types=[
          pltpu.VMEM((_NR, 2 * _CW), jnp.int32),
          pltpu.VMEM((_NR, 2 * _CW), jnp.int32),
          pltpu.VMEM((_CW, d), jnp.float32),
          pltpu.VMEM((_CW, d), jnp.float32),
          pltpu.SemaphoreType.DMA,
          pltpu.SemaphoreType.DMA,
          pltpu.SemaphoreType.DMA,
          pltpu.SemaphoreType.DMA,
          pltpu.SemaphoreType.DMA,
          pltpu.SemaphoreType.DMA,
          pltpu.VMEM_SHARED((npad, d), jnp.float32),
          pltpu.VMEM_SHARED((nhalf, d), jnp.float32),
      ],
  )
  return fn(x, sd0, sd1, zrow)


def _count_body(nch, npad, d,
                dstp_hbm, zcnt_hbm, ones_hbm,
                cntp_hbm,
                dst_t, ones_v, cnt_sh):
  c = lax.axis_index("c")
  s = lax.axis_index("s")
  wid = c * _NS + s
  zrows = npad // _NS

  pltpu.sync_copy(dstp_hbm.at[pl.ds(wid * nch, nch)], dst_t)
  pltpu.sync_copy(ones_hbm, ones_v)
  pltpu.sync_copy(zcnt_hbm, cnt_sh.at[pl.ds(s * zrows, zrows)])
  plsc.subcore_barrier()

  def chunk(j, carry):
    pltpu.sync_copy(ones_v, cnt_sh.at[dst_t.at[j]], add=True)
    return carry

  lax.fori_loop(0, nch, chunk, 0)
  plsc.subcore_barrier()
  pltpu.sync_copy(cnt_sh.at[pl.ds(s * zrows, zrows)],
                  cntp_hbm.at[c, pl.ds(s * zrows, zrows)])


@functools.partial(jax.jit, static_argnums=(1, 2))
def _sc_counts(dstp, n, d):
  nt = _NC * _NS
  nch = dstp.shape[0] // nt
  npad = (n + 128) // 128 * 128
  zcnt = jnp.zeros((npad // _NS, d), jnp.float32)
  ones = jnp.ones((_LW, d), jnp.float32)
  mesh = plsc.VectorSubcoreMesh(core_axis_name="c", subcore_axis_name="s")
  fn = pl.kernel(
      functools.partial(_count_body, nch, npad, d),
      out_type=jax.ShapeDtypeStruct((_NC, npad, d), jnp.float32),
      mesh=mesh,
      scratch_types=[
          pltpu.VMEM((nch, _LW), jnp.int32),
          pltpu.VMEM((_LW, d), jnp.float32),
          pltpu.VMEM_SHARED((npad, d), jnp.float32),
      ],
  )
  return fn(dstp, zcnt, ones)


def _phase_body(final, p_ref, cnt_ref, x_ref, wl_ref, wr_ref, b_ref, o_ref):
  s = p_ref[0] + p_ref[1]
  cnt = cnt_ref[0][:, :1] + cnt_ref[1][:, :1]
  mean = s / jnp.maximum(cnt, 1.0)
  h = (jnp.dot(mean, wl_ref[...], preferred_element_type=jnp.float32)
       + jnp.dot(x_ref[...], wr_ref[...], preferred_element_type=jnp.float32)
       + b_ref[...])
  if final:
    nrm = jnp.sqrt(jnp.sum(h * h, axis=1, keepdims=True))
    o_ref[...] = h / jnp.maximum(nrm, 1e-12)
  else:
    o_ref[...] = jnp.where(h >= 0, h, 0.01 * h)


@functools.partial(jax.jit, static_argnums=(6,))
def _tc_phase(part, cntp, x, wlt, wrt, b, final):
  n, d = x.shape
  r = 1000
  grid = (n // r,)
  return pl.pallas_call(
      functools.partial(_phase_body, final),
      grid=grid,
      in_specs=[
          pl.BlockSpec((_NC, r, d), lambda i: (0, i, 0)),
          pl.BlockSpec((_NC, r, d), lambda i: (0, i, 0)),
          pl.BlockSpec((r, d), lambda i: (i, 0)),
          pl.BlockSpec((d, d), lambda i: (0, 0)),
          pl.BlockSpec((d, d), lambda i: (0, 0)),
          pl.BlockSpec((1, d), lambda i: (0, 0)),
      ],
      out_specs=pl.BlockSpec((r, d), lambda i: (i, 0)),
      out_shape=jax.ShapeDtypeStruct((n, d), jnp.float32),
  )(part, cntp, x, wlt, wrt, b)


def kernel(x, edge_indices, Wl1, Wr1, b1, Wl2, Wr2, b2):
  n, d = x.shape
  e = edge_indices.shape[1]
  nt = _NC * _NS
  nhalf = n // 2

  nch = -(-e // (_LW * nt))
  nch = (nch + 7) // 8 * 8
  epad = nch * _LW * nt
  dst128 = jnp.concatenate(
      [edge_indices[1], jnp.full((epad - e,), n, jnp.int32)]).reshape(-1, _LW)

  m = -(-e // (_CW * nt))
  m = (m + _NR - 1) // _NR * _NR
  ep2 = m * _CW * nt
  srcf = jnp.concatenate([edge_indices[0], jnp.full((ep2 - e,), n, jnp.int32)])
  dstf = jnp.concatenate([edge_indices[1], jnp.zeros((ep2 - e,), jnp.int32)])
  sd = []
  for h in (0, 1):
    lo = h * nhalf
    inh = (srcf >= lo) & (srcf < lo + nhalf)
    sp = jnp.where(inh, srcf - lo, 0).reshape(-1, 1, _CW)
    dp = jnp.where(inh, dstf, n).reshape(-1, 1, _CW)
    sd.append(jnp.concatenate([dp, sp], axis=1).reshape(-1, 2 * _CW))

  cp = _sc_counts(dst128, n, d)
  p1 = _sc_segsum(x, sd[0], sd[1], n, d)
  h = _tc_phase(p1, cp, x, Wl1.T, Wr1.T, b1.reshape(1, d), False)
  p2 = _sc_segsum(h, sd[0], sd[1], n, d)
  return _tc_phase(p2, cp, h, Wl2.T, Wr2.T, b2.reshape(1, d), True)
